# baseline (speedup 1.0000x reference)
import functools

import jax
import jax.numpy as jnp
from jax import lax
from jax.experimental import pallas as pl
from jax.experimental.pallas import tpu as pltpu
from jax.experimental.pallas import tpu_sc as plsc

_NC = 2
_NS = 16
_NW = _NC * _NS
_CR = 16
_K = 2


@functools.cache
def _make_gather(bx: int, f: int, d: int):
    xr = bx // _NW
    nchunk = xr // _CR
    ngroups = nchunk // _K
    assert xr % _CR == 0 and nchunk % _K == 0 and ngroups % 2 == 0 and ngroups >= 4
    mesh = plsc.VectorSubcoreMesh(core_axis_name="c", subcore_axis_name="s")

    @functools.partial(
        pl.kernel,
        mesh=mesh,
        out_type=jax.ShapeDtypeStruct((bx, f, d), jnp.float32),
        scratch_types=[
            pltpu.VMEM((xr, f), jnp.int32),
            pltpu.VMEM((2, _K, _CR, f, d), jnp.float32),
            pltpu.SemaphoreType.DMA,
            pltpu.SemaphoreType.DMA,
            pltpu.SemaphoreType.DMA,
            pltpu.SemaphoreType.DMA,
        ],
        compiler_params=pltpu.CompilerParams(use_tc_tiling_on_sc=False),
    )
    def k(table_hbm, x_hbm, out_hbm, idx_v, rows_v, gsa, gsb, ssa, ssb):
        wid = lax.axis_index("s") * _NC + lax.axis_index("c")
        base = wid * xr
        pltpu.sync_copy(x_hbm.at[pl.ds(base, xr)], idx_v)

        def gather_descs(grp, bufset, b, sem):
            return [
                pltpu.make_async_copy(
                    table_hbm.at[idx_v.at[(grp * _K + b) * _CR + r]],
                    rows_v.at[bufset, b, r],
                    sem,
                )
                for r in range(_CR)
            ]

        def store_desc(grp, bufset, b, sem):
            return pltpu.make_async_copy(
                rows_v.at[bufset, b],
                out_hbm.at[pl.ds(base + (grp * _K + b) * _CR, _CR)],
                sem,
            )

        def _descs(desc_fn, grp, bufset, sem):
            out = []
            for b in range(_K):
                d_ = desc_fn(grp, bufset, b, sem)
                out.extend(d_ if isinstance(d_, list) else [d_])
            return out

        def fire(desc_fn, grp, bufset, sem):
            for d_ in _descs(desc_fn, grp, bufset, sem):
                d_.start()

        def drain(desc_fn, grp, bufset, sem):
            for d_ in _descs(desc_fn, grp, bufset, sem):
                d_.wait()

        fire(gather_descs, 0, 0, gsa)
        fire(gather_descs, 1, 1, gsb)
        drain(gather_descs, 0, 0, gsa)
        fire(store_desc, 0, 0, ssa)

        def pair_body(p, carry):
            drain(store_desc, 2 * p, 0, ssa)
            fire(gather_descs, 2 * p + 2, 0, gsa)
            drain(gather_descs, 2 * p + 1, 1, gsb)
            fire(store_desc, 2 * p + 1, 1, ssb)
            drain(store_desc, 2 * p + 1, 1, ssb)
            fire(gather_descs, 2 * p + 3, 1, gsb)
            drain(gather_descs, 2 * p + 2, 0, gsa)
            fire(store_desc, 2 * p + 2, 0, ssa)
            return carry

        lax.fori_loop(0, ngroups // 2 - 1, pair_body, 0)

        drain(store_desc, ngroups - 2, 0, ssa)
        drain(gather_descs, ngroups - 1, 1, gsb)
        fire(store_desc, ngroups - 1, 1, ssb)
        drain(store_desc, ngroups - 1, 1, ssb)

    return k


def kernel(x, table):
    bx, f = x.shape
    n, d = table.shape
    return _make_gather(bx, f, d)(table, x)
